# Initial kernel scaffold; baseline (speedup 1.0000x reference)
import jax
import jax.numpy as jnp
from jax.experimental import pallas as pl


def kernel(boxes, scores, deltas):
    raise NotImplementedError("write your pallas kernel here")


# speedup vs baseline: 142.3932x; 142.3932x over previous
import jax
import jax.numpy as jnp
from jax.experimental import pallas as pl

_N = 20000
_ROWS = 157
_PAD = _ROWS * 128
_PRE_K = 2000
_TOP_K = 100
_NMS_T = 0.3
_THR = 0.1


def _nms_kernel(bx_ref, sc_ref, dl_ref, out_ref):
    b = bx_ref[...]
    q = jnp.clip(jnp.round(b * 4.0), -32768.0, 32767.0) * 0.25
    x1b, y1b, x2b, y2b = q[0], q[1], q[2], q[3]
    w = x2b - x1b
    h = y2b - y1b
    cx = x1b + 0.5 * w
    cy = y1b + 0.5 * h
    d = dl_ref[...]
    pcx = d[0] * w + cx
    pcy = d[1] * h + cy
    pw = w * jnp.exp(jnp.clip(d[2], -10.0, 10.0))
    ph = h * jnp.exp(jnp.clip(d[3], -10.0, 10.0))
    X1 = jnp.clip(pcx - 0.5 * pw, 0.0, 1023.0)
    Y1 = jnp.clip(pcy - 0.5 * ph, 0.0, 1023.0)
    X2 = jnp.clip(pcx + 0.5 * pw, 0.0, 1023.0)
    Y2 = jnp.clip(pcy + 0.5 * ph, 0.0, 1023.0)
    AREA = jnp.maximum(X2 - X1, 0.0) * jnp.maximum(Y2 - Y1, 0.0)

    sc = sc_ref[...]
    S0 = jnp.where(sc >= _THR, sc, -1.0)
    IDX = (jax.lax.broadcasted_iota(jnp.int32, (_ROWS, 128), 0) * 128
           + jax.lax.broadcasted_iota(jnp.int32, (_ROWS, 128), 1))
    lane = jax.lax.broadcasted_iota(jnp.int32, (1, 128), 1)
    zrow = jnp.zeros((1, 128), jnp.float32)

    def cond(st):
        return jnp.logical_not(st[0])

    def body(st):
        done, k, p, S, kx1, ky1, kx2, ky2, ks, ka = st
        m = jnp.max(S)
        j = jnp.min(jnp.where(S == m, IDX, jnp.int32(2 ** 30)))
        msk = IDX == j
        S = jnp.where(msk, -2.0, S)
        valid = m >= _THR
        mf = msk.astype(jnp.float32)
        cx1 = jnp.sum(mf * X1)
        cy1 = jnp.sum(mf * Y1)
        cx2 = jnp.sum(mf * X2)
        cy2 = jnp.sum(mf * Y2)
        car = jnp.sum(mf * AREA)
        ix1 = jnp.maximum(kx1, cx1)
        iy1 = jnp.maximum(ky1, cy1)
        ix2 = jnp.minimum(kx2, cx2)
        iy2 = jnp.minimum(ky2, cy2)
        inter = jnp.maximum(ix2 - ix1, 0.0) * jnp.maximum(iy2 - iy1, 0.0)
        iou = inter / jnp.maximum(ka + car - inter, 1e-9)
        sup = jnp.any((lane < k) & (iou > _NMS_T))
        keep = valid & jnp.logical_not(sup)
        at_k = (lane == k) & keep
        kx1 = jnp.where(at_k, cx1, kx1)
        ky1 = jnp.where(at_k, cy1, ky1)
        kx2 = jnp.where(at_k, cx2, kx2)
        ky2 = jnp.where(at_k, cy2, ky2)
        ks = jnp.where(at_k, m, ks)
        ka = jnp.where(at_k, car, ka)
        k = k + keep.astype(jnp.int32)
        p = p + 1
        done = jnp.logical_not(valid) | (k >= _TOP_K) | (p >= _PRE_K)
        return done, k, p, S, kx1, ky1, kx2, ky2, ks, ka

    st0 = (jnp.bool_(False), jnp.int32(0), jnp.int32(0), S0,
           zrow, zrow, zrow, zrow, zrow, zrow)
    _, k, _, _, kx1, ky1, kx2, ky2, ks, ka = jax.lax.while_loop(cond, body, st0)

    live = lane < k
    neg = jnp.full((1, 128), -1.0, jnp.float32)
    out_ref[0:1, :] = jnp.where(live, kx1, neg)
    out_ref[1:2, :] = jnp.where(live, ky1, neg)
    out_ref[2:3, :] = jnp.where(live, kx2, neg)
    out_ref[3:4, :] = jnp.where(live, ky2, neg)
    out_ref[4:5, :] = jnp.where(live, ks, neg)
    out_ref[5:6, :] = jnp.where(live, jnp.ones((1, 128), jnp.float32), neg)
    out_ref[6:8, :] = jnp.full((2, 128), -1.0, jnp.float32)


def kernel(boxes, scores, deltas):
    sc = scores.reshape(_N, 2)[:, 1]
    dl = deltas.reshape(_N, 2, 4)[:, 1, :]
    pad = _PAD - _N
    bx = jnp.pad(boxes, ((0, pad), (0, 0))).T.reshape(4, _ROWS, 128)
    dlp = jnp.pad(dl, ((0, pad), (0, 0))).T.reshape(4, _ROWS, 128)
    scp = jnp.pad(sc, (0, pad), constant_values=-1.0).reshape(_ROWS, 128)
    out = pl.pallas_call(
        _nms_kernel,
        out_shape=jax.ShapeDtypeStruct((8, 128), jnp.float32),
    )(bx, scp, dlp)
    return out[:6, :_TOP_K].T[None, :, :]


# speedup vs baseline: 177.9531x; 1.2497x over previous
import functools

import jax
import jax.numpy as jnp
from jax import lax
from jax.experimental import pallas as pl
from jax.experimental.pallas import tpu as pltpu
from jax.experimental.pallas import tpu_sc as plsc

_N = 20000
_PADN = 20480
_PER_TILE = 640
_NBLK = 1280
_NGRP = 80
_PRE_K = 2000
_TOP_K = 100
_KPAD = 112
_NMS_T = 0.3
_THR = 0.1
_F32 = jnp.float32

@functools.cache
def _mesh():
    return plsc.VectorSubcoreMesh(core_axis_name="c", subcore_axis_name="s")


def _round_half_even_nonneg(x):
    i = x.astype(jnp.int32)
    f = x - i.astype(_F32)
    up = (f > 0.5) | ((f == 0.5) & ((i & 1) == 1))
    return jnp.where(up, i + 1, i).astype(_F32)


def _decode_body(x1_h, y1_h, x2_h, y2_h, d0_h, d1_h, d2_h, d3_h, sc_h, dec_h,
                 x1_v, y1_v, x2_v, y2_v, d0_v, d1_v, d2_v, d3_v, sc_v,
                 ox1, oy1, ox2, oy2, oar, osc):
    wid = lax.axis_index("s") * 2 + lax.axis_index("c")
    base = wid * _PER_TILE
    sl_h = pl.ds(base, _PER_TILE)
    pltpu.sync_copy(x1_h.at[sl_h], x1_v)
    pltpu.sync_copy(y1_h.at[sl_h], y1_v)
    pltpu.sync_copy(x2_h.at[sl_h], x2_v)
    pltpu.sync_copy(y2_h.at[sl_h], y2_v)
    pltpu.sync_copy(d0_h.at[sl_h], d0_v)
    pltpu.sync_copy(d1_h.at[sl_h], d1_v)
    pltpu.sync_copy(d2_h.at[sl_h], d2_v)
    pltpu.sync_copy(d3_h.at[sl_h], d3_v)
    pltpu.sync_copy(sc_h.at[sl_h], sc_v)

    def step(i, _):
        sl = pl.ds(i * 16, 16)
        qx1 = jnp.clip(_round_half_even_nonneg(x1_v[sl] * 4.0), -32768.0, 32767.0) * 0.25
        qy1 = jnp.clip(_round_half_even_nonneg(y1_v[sl] * 4.0), -32768.0, 32767.0) * 0.25
        qx2 = jnp.clip(_round_half_even_nonneg(x2_v[sl] * 4.0), -32768.0, 32767.0) * 0.25
        qy2 = jnp.clip(_round_half_even_nonneg(y2_v[sl] * 4.0), -32768.0, 32767.0) * 0.25
        w = qx2 - qx1
        h = qy2 - qy1
        cx = qx1 + 0.5 * w
        cy = qy1 + 0.5 * h
        pcx = d0_v[sl] * w + cx
        pcy = d1_v[sl] * h + cy
        pw = w * jnp.exp(jnp.clip(d2_v[sl], -10.0, 10.0))
        ph = h * jnp.exp(jnp.clip(d3_v[sl], -10.0, 10.0))
        rx1 = jnp.clip(pcx - 0.5 * pw, 0.0, 1023.0)
        ry1 = jnp.clip(pcy - 0.5 * ph, 0.0, 1023.0)
        rx2 = jnp.clip(pcx + 0.5 * pw, 0.0, 1023.0)
        ry2 = jnp.clip(pcy + 0.5 * ph, 0.0, 1023.0)
        ox1[sl] = rx1
        oy1[sl] = ry1
        ox2[sl] = rx2
        oy2[sl] = ry2
        oar[sl] = jnp.maximum(rx2 - rx1, 0.0) * jnp.maximum(ry2 - ry1, 0.0)
        s = sc_v[sl]
        osc[sl] = jnp.where(s >= _THR, s, -1.0)
        return _

    lax.fori_loop(0, _PER_TILE // 16, step, 0)

    pltpu.sync_copy(ox1, dec_h.at[0, sl_h])
    pltpu.sync_copy(oy1, dec_h.at[1, sl_h])
    pltpu.sync_copy(ox2, dec_h.at[2, sl_h])
    pltpu.sync_copy(oy2, dec_h.at[3, sl_h])
    pltpu.sync_copy(oar, dec_h.at[4, sl_h])
    pltpu.sync_copy(osc, dec_h.at[5, sl_h])


@functools.cache
def _decode():
    return pl.kernel(
        _decode_body,
        jax.ShapeDtypeStruct((6, _PADN), _F32),
        mesh=_mesh(),
        compiler_params=pltpu.CompilerParams(needs_layout_passes=False),
        scratch_types=[pltpu.VMEM((_PER_TILE,), _F32) for _ in range(15)],
    )


def _nms_body(dec_h, out_h,
              X1v, Y1v, X2v, Y2v, ARv, Sv, L1v, L2v,
              kx1, ky1, kx2, ky2, ks, ka, outv):
    c = lax.axis_index("c")
    s = lax.axis_index("s")

    @pl.when((c == 0) & (s == 0))
    def _run():
        pltpu.sync_copy(dec_h.at[0], X1v)
        pltpu.sync_copy(dec_h.at[1], Y1v)
        pltpu.sync_copy(dec_h.at[2], X2v)
        pltpu.sync_copy(dec_h.at[3], Y2v)
        pltpu.sync_copy(dec_h.at[4], ARv)
        pltpu.sync_copy(dec_h.at[5], Sv)

        iota = lax.iota(jnp.int32, 16)
        big = jnp.int32(1 << 30)

        def l1_step(cc, _):
            acc = jnp.full((16,), -2.0, _F32)
            for l in range(16):
                m = jnp.max(Sv[pl.ds(cc * 256 + l * 16, 16)])
                acc = jnp.where(iota == l, m, acc)
            L1v[pl.ds(cc * 16, 16)] = acc
            return _

        lax.fori_loop(0, _NGRP, l1_step, 0)

        def l2_step(gg, _):
            acc = jnp.full((16,), -2.0, _F32)
            for l in range(16):
                m = jnp.max(L1v[pl.ds(gg * 256 + l * 16, 16)])
                acc = jnp.where(iota == l, m, acc)
            L2v[pl.ds(gg * 16, 16)] = acc
            return _

        lax.fori_loop(0, _NGRP // 16, l2_step, 0)

        def cond(st):
            return jnp.logical_not(st[0])

        def body(st):
            done, k, p = st
            gm = jnp.float32(-2.0)
            for g8 in range(5):
                gm = jnp.maximum(gm, jnp.max(L2v[pl.ds(g8 * 16, 16)]))
            g = big
            for g8 in range(5):
                v = L2v[pl.ds(g8 * 16, 16)]
                g = jnp.minimum(g, jnp.min(jnp.where(v == gm, g8 * 16 + iota, big)))
            vL1 = L1v[pl.ds(g * 16, 16)]
            b = jnp.min(jnp.where(vL1 == gm, g * 16 + iota, big))
            vS = Sv[pl.ds(b * 16, 16)]
            l = jnp.min(jnp.where(vS == gm, iota, big))
            lane = iota == l
            valid = gm >= _THR

            cx1 = jnp.sum(jnp.where(lane, X1v[pl.ds(b * 16, 16)], 0.0))
            cy1 = jnp.sum(jnp.where(lane, Y1v[pl.ds(b * 16, 16)], 0.0))
            cx2 = jnp.sum(jnp.where(lane, X2v[pl.ds(b * 16, 16)], 0.0))
            cy2 = jnp.sum(jnp.where(lane, Y2v[pl.ds(b * 16, 16)], 0.0))
            car = jnp.sum(jnp.where(lane, ARv[pl.ds(b * 16, 16)], 0.0))

            vS2 = jnp.where(lane, -2.0, vS)
            Sv[pl.ds(b * 16, 16)] = vS2
            vL1n = jnp.where(iota == b - g * 16, jnp.max(vS2), vL1)
            L1v[pl.ds(g * 16, 16)] = vL1n
            g8d = g // 16
            vL2 = L2v[pl.ds(g8d * 16, 16)]
            L2v[pl.ds(g8d * 16, 16)] = jnp.where(
                iota == g - g8d * 16, jnp.max(vL1n), vL2)

            sup = jnp.float32(0.0)
            for c7 in range(7):
                slk = pl.ds(c7 * 16, 16)
                ix1 = jnp.maximum(kx1[slk], cx1)
                iy1 = jnp.maximum(ky1[slk], cy1)
                ix2 = jnp.minimum(kx2[slk], cx2)
                iy2 = jnp.minimum(ky2[slk], cy2)
                inter = jnp.maximum(ix2 - ix1, 0.0) * jnp.maximum(iy2 - iy1, 0.0)
                iou = inter / jnp.maximum(ka[slk] + car - inter, 1e-9)
                iou = jnp.where(c7 * 16 + iota < k, iou, 0.0)
                sup = jnp.maximum(sup, jnp.max(iou))
            keep = valid & (sup <= _NMS_T)

            kc = k // 16
            slot = (iota == k - kc * 16) & keep
            slk = pl.ds(kc * 16, 16)
            kx1[slk] = jnp.where(slot, cx1, kx1[slk])
            ky1[slk] = jnp.where(slot, cy1, ky1[slk])
            kx2[slk] = jnp.where(slot, cx2, kx2[slk])
            ky2[slk] = jnp.where(slot, cy2, ky2[slk])
            ks[slk] = jnp.where(slot, gm, ks[slk])
            ka[slk] = jnp.where(slot, car, ka[slk])

            k = jnp.where(keep, k + 1, k)
            p = p + 1
            done = (jnp.logical_not(valid) | (k >= _TOP_K) | (p >= _PRE_K))
            return done, k, p

        _, kfin, _ = lax.while_loop(
            cond, body, (jnp.bool_(False), jnp.int32(0), jnp.int32(0)))

        neg = jnp.full((16,), -1.0, _F32)
        for r, arr in enumerate([kx1, ky1, kx2, ky2, ks]):
            for c7 in range(7):
                live = c7 * 16 + iota < kfin
                outv[pl.ds(r * _KPAD + c7 * 16, 16)] = jnp.where(
                    live, arr[pl.ds(c7 * 16, 16)], neg)
        for c7 in range(7):
            live = c7 * 16 + iota < kfin
            outv[pl.ds(5 * _KPAD + c7 * 16, 16)] = jnp.where(live, 1.0, neg)
        pltpu.sync_copy(outv, out_h)


@functools.cache
def _nms():
    return pl.kernel(
        _nms_body,
        jax.ShapeDtypeStruct((6 * _KPAD,), _F32),
        mesh=_mesh(),
        compiler_params=pltpu.CompilerParams(needs_layout_passes=False),
        scratch_types=(
            [pltpu.VMEM((_PADN,), _F32) for _ in range(6)]
            + [pltpu.VMEM((_NBLK,), _F32), pltpu.VMEM((_NGRP,), _F32)]
            + [pltpu.VMEM((_KPAD,), _F32) for _ in range(6)]
            + [pltpu.VMEM((6 * _KPAD,), _F32)]),
    )


def kernel(boxes, scores, deltas):
    pad = _PADN - _N
    sc = jnp.pad(scores.reshape(_N, 2)[:, 1], (0, pad), constant_values=-1.0)
    dl = jnp.pad(deltas.reshape(_N, 2, 4)[:, 1, :], ((0, pad), (0, 0)))
    bx = jnp.pad(boxes, ((0, pad), (0, 0)))
    dec = _decode()(bx[:, 0], bx[:, 1], bx[:, 2], bx[:, 3],
                    dl[:, 0], dl[:, 1], dl[:, 2], dl[:, 3], sc)
    out = _nms()(dec)
    return out.reshape(6, _KPAD)[:, :_TOP_K].T[None, :, :]


# speedup vs baseline: 183.9672x; 1.0338x over previous
import functools

import jax
import jax.numpy as jnp
from jax import lax
from jax.experimental import pallas as pl
from jax.experimental.pallas import tpu as pltpu
from jax.experimental.pallas import tpu_sc as plsc

_N = 20000
_PADN = 20480
_PER_TILE = 640
_NBLK = 1280
_NGRP = 80
_PRE_K = 2000
_TOP_K = 100
_KPAD = 112
_NMS_T = 0.3
_THR = 0.1
_F32 = jnp.float32
_DROWS = 7

@functools.cache
def _mesh():
    return plsc.VectorSubcoreMesh(core_axis_name="c", subcore_axis_name="s")


def _round_half_even_nonneg(x):
    i = x.astype(jnp.int32)
    f = x - i.astype(_F32)
    up = (f > 0.5) | ((f == 0.5) & ((i & 1) == 1))
    return jnp.where(up, i + 1, i).astype(_F32)


def _decode_body(x1_h, y1_h, x2_h, y2_h, d0_h, d1_h, d2_h, d3_h, sc_h, dec_h, l1_h,
                 x1_v, y1_v, x2_v, y2_v, d0_v, d1_v, d2_v, d3_v, sc_v,
                 ox1, oy1, ox2, oy2, oar, osc, ol1):
    wid = lax.axis_index("s") * 2 + lax.axis_index("c")
    base = pl.multiple_of(wid * _PER_TILE, 8)
    sl_h = pl.ds(base, _PER_TILE)
    pltpu.sync_copy(x1_h.at[sl_h], x1_v)
    pltpu.sync_copy(y1_h.at[sl_h], y1_v)
    pltpu.sync_copy(x2_h.at[sl_h], x2_v)
    pltpu.sync_copy(y2_h.at[sl_h], y2_v)
    pltpu.sync_copy(d0_h.at[sl_h], d0_v)
    pltpu.sync_copy(d1_h.at[sl_h], d1_v)
    pltpu.sync_copy(d2_h.at[sl_h], d2_v)
    pltpu.sync_copy(d3_h.at[sl_h], d3_v)
    pltpu.sync_copy(sc_h.at[sl_h], sc_v)

    def step(i, _):
        sl = pl.ds(i * 16, 16)
        qx1 = jnp.clip(_round_half_even_nonneg(x1_v[sl] * 4.0), -32768.0, 32767.0) * 0.25
        qy1 = jnp.clip(_round_half_even_nonneg(y1_v[sl] * 4.0), -32768.0, 32767.0) * 0.25
        qx2 = jnp.clip(_round_half_even_nonneg(x2_v[sl] * 4.0), -32768.0, 32767.0) * 0.25
        qy2 = jnp.clip(_round_half_even_nonneg(y2_v[sl] * 4.0), -32768.0, 32767.0) * 0.25
        w = qx2 - qx1
        h = qy2 - qy1
        cx = qx1 + 0.5 * w
        cy = qy1 + 0.5 * h
        pcx = d0_v[sl] * w + cx
        pcy = d1_v[sl] * h + cy
        pw = w * jnp.exp(jnp.clip(d2_v[sl], -10.0, 10.0))
        ph = h * jnp.exp(jnp.clip(d3_v[sl], -10.0, 10.0))
        rx1 = jnp.clip(pcx - 0.5 * pw, 0.0, 1023.0)
        ry1 = jnp.clip(pcy - 0.5 * ph, 0.0, 1023.0)
        rx2 = jnp.clip(pcx + 0.5 * pw, 0.0, 1023.0)
        ry2 = jnp.clip(pcy + 0.5 * ph, 0.0, 1023.0)
        ox1[sl] = rx1
        oy1[sl] = ry1
        ox2[sl] = rx2
        oy2[sl] = ry2
        oar[sl] = jnp.maximum(rx2 - rx1, 0.0) * jnp.maximum(ry2 - ry1, 0.0)
        s = sc_v[sl]
        osc[sl] = jnp.where(s >= _THR, s, -1.0)
        return _

    lax.fori_loop(0, _PER_TILE // 16, step, 0)

    iota = lax.iota(jnp.int32, 16)
    def l1_step(c2, _):
        acc = jnp.full((16,), -2.0, _F32)
        for l in range(16):
            blk = c2 * 16 + l
            if blk < 40:
                m = jnp.max(osc[pl.ds(blk * 16, 16)])
                acc = jnp.where(iota == l, m, acc)
        ol1[pl.ds(c2 * 16, 16)] = acc
        return _

    for c2 in range(3):
        l1_step(c2, 0)

    pltpu.sync_copy(ox1, dec_h.at[0, sl_h])
    pltpu.sync_copy(oy1, dec_h.at[1, sl_h])
    pltpu.sync_copy(ox2, dec_h.at[2, sl_h])
    pltpu.sync_copy(oy2, dec_h.at[3, sl_h])
    pltpu.sync_copy(oar, dec_h.at[4, sl_h])
    pltpu.sync_copy(osc, dec_h.at[5, sl_h])
    pltpu.sync_copy(ol1.at[pl.ds(0, 40)], l1_h.at[pl.ds(pl.multiple_of(wid * 40, 8), 40)])


@functools.cache
def _decode():
    return pl.kernel(
        _decode_body,
        (jax.ShapeDtypeStruct((6, _PADN), _F32),
         jax.ShapeDtypeStruct((_NBLK,), _F32)),
        mesh=_mesh(),
        compiler_params=pltpu.CompilerParams(needs_layout_passes=False),
        scratch_types=[pltpu.VMEM((_PER_TILE,), _F32) for _ in range(15)]
        + [pltpu.VMEM((48,), _F32)],
    )


def _nms_body(dec_h, l1_h, out_h,
              X1v, Y1v, X2v, Y2v, ARv, Sv, L1v, L2v,
              kx1, ky1, kx2, ky2, ks, ka, outv):
    c = lax.axis_index("c")
    s = lax.axis_index("s")

    @pl.when((c == 0) & (s == 0))
    def _run():
        pltpu.sync_copy(dec_h.at[0], X1v)
        pltpu.sync_copy(dec_h.at[1], Y1v)
        pltpu.sync_copy(dec_h.at[2], X2v)
        pltpu.sync_copy(dec_h.at[3], Y2v)
        pltpu.sync_copy(dec_h.at[4], ARv)
        pltpu.sync_copy(dec_h.at[5], Sv)
        pltpu.sync_copy(l1_h, L1v)

        iota = lax.iota(jnp.int32, 16)
        big = jnp.int32(1 << 30)

        def l2_step(gg, _):
            acc = jnp.full((16,), -2.0, _F32)
            for l in range(16):
                m = jnp.max(L1v[pl.ds(gg * 256 + l * 16, 16)])
                acc = jnp.where(iota == l, m, acc)
            L2v[pl.ds(gg * 16, 16)] = acc
            return _

        lax.fori_loop(0, _NGRP // 16, l2_step, 0)

        def cond(st):
            return jnp.logical_not(st[0])

        def body(st):
            done, k, p = st
            v0 = L2v[pl.ds(0, 16)]
            v1 = L2v[pl.ds(16, 16)]
            v2 = L2v[pl.ds(32, 16)]
            v3 = L2v[pl.ds(48, 16)]
            v4 = L2v[pl.ds(64, 16)]
            gm = jnp.max(jnp.maximum(jnp.maximum(jnp.maximum(v0, v1),
                                                 jnp.maximum(v2, v3)), v4))
            c0 = jnp.where(v0 == gm, iota, big)
            c1 = jnp.where(v1 == gm, 16 + iota, big)
            c2 = jnp.where(v2 == gm, 32 + iota, big)
            c3 = jnp.where(v3 == gm, 48 + iota, big)
            c4 = jnp.where(v4 == gm, 64 + iota, big)
            g = jnp.min(jnp.minimum(jnp.minimum(jnp.minimum(c0, c1),
                                                jnp.minimum(c2, c3)), c4))
            vL1 = L1v[pl.ds(g * 16, 16)]
            b = jnp.min(jnp.where(vL1 == gm, g * 16 + iota, big))
            vS = Sv[pl.ds(b * 16, 16)]
            l = jnp.min(jnp.where(vS == gm, iota, big))
            lane = iota == l
            valid = gm >= _THR

            jvec = jnp.zeros((16,), jnp.int32) + (b * 16 + l)
            cx1 = plsc.load_gather(X1v, [jvec])
            cy1 = plsc.load_gather(Y1v, [jvec])
            cx2 = plsc.load_gather(X2v, [jvec])
            cy2 = plsc.load_gather(Y2v, [jvec])
            car = plsc.load_gather(ARv, [jvec])

            vS2 = jnp.where(lane, -2.0, vS)
            Sv[pl.ds(b * 16, 16)] = vS2
            vL1n = jnp.where(iota == b - g * 16, jnp.max(vS2), vL1)
            L1v[pl.ds(g * 16, 16)] = vL1n
            g8d = g // 16
            vL2 = L2v[pl.ds(g8d * 16, 16)]
            L2v[pl.ds(g8d * 16, 16)] = jnp.where(
                iota == g - g8d * 16, jnp.max(vL1n), vL2)

            supv = jnp.zeros((16,), _F32)
            for c7 in range(7):
                slk = pl.ds(c7 * 16, 16)
                ix1 = jnp.maximum(kx1[slk], cx1)
                iy1 = jnp.maximum(ky1[slk], cy1)
                ix2 = jnp.minimum(kx2[slk], cx2)
                iy2 = jnp.minimum(ky2[slk], cy2)
                inter = jnp.maximum(ix2 - ix1, 0.0) * jnp.maximum(iy2 - iy1, 0.0)
                iou = inter / jnp.maximum(ka[slk] + car - inter, 1e-9)
                supv = jnp.maximum(supv, jnp.where(c7 * 16 + iota < k, iou, 0.0))
            keep = valid & (jnp.max(supv) <= _NMS_T)

            kc = k // 16
            slot = (iota == k - kc * 16) & keep
            slk = pl.ds(kc * 16, 16)
            kx1[slk] = jnp.where(slot, cx1, kx1[slk])
            ky1[slk] = jnp.where(slot, cy1, ky1[slk])
            kx2[slk] = jnp.where(slot, cx2, kx2[slk])
            ky2[slk] = jnp.where(slot, cy2, ky2[slk])
            ks[slk] = jnp.where(slot, gm, ks[slk])
            ka[slk] = jnp.where(slot, car, ka[slk])

            k = jnp.where(keep, k + 1, k)
            p = p + 1
            done = (jnp.logical_not(valid) | (k >= _TOP_K) | (p >= _PRE_K))
            return done, k, p

        _, kfin, _ = lax.while_loop(
            cond, body, (jnp.bool_(False), jnp.int32(0), jnp.int32(0)))

        neg = jnp.full((16,), -1.0, _F32)
        for r, arr in enumerate([kx1, ky1, kx2, ky2, ks]):
            for c7 in range(7):
                live = c7 * 16 + iota < kfin
                outv[pl.ds(r * _KPAD + c7 * 16, 16)] = jnp.where(
                    live, arr[pl.ds(c7 * 16, 16)], neg)
        for c7 in range(7):
            live = c7 * 16 + iota < kfin
            outv[pl.ds(5 * _KPAD + c7 * 16, 16)] = jnp.where(live, 1.0, neg)
        pltpu.sync_copy(outv, out_h)


@functools.cache
def _nms():
    return pl.kernel(
        _nms_body,
        jax.ShapeDtypeStruct((6 * _KPAD,), _F32),
        mesh=_mesh(),
        compiler_params=pltpu.CompilerParams(needs_layout_passes=False),
        scratch_types=(
            [pltpu.VMEM((_PADN,), _F32) for _ in range(6)]
            + [pltpu.VMEM((_NBLK,), _F32), pltpu.VMEM((_NGRP,), _F32)]
            + [pltpu.VMEM((_KPAD,), _F32) for _ in range(6)]
            + [pltpu.VMEM((6 * _KPAD,), _F32)]),
    )


def kernel(boxes, scores, deltas):
    pad = _PADN - _N
    sc = jnp.pad(scores.reshape(_N, 2)[:, 1], (0, pad), constant_values=-1.0)
    dl = jnp.pad(deltas.reshape(_N, 2, 4)[:, 1, :], ((0, pad), (0, 0)))
    bx = jnp.pad(boxes, ((0, pad), (0, 0)))
    dec, l1 = _decode()(bx[:, 0], bx[:, 1], bx[:, 2], bx[:, 3],
                        dl[:, 0], dl[:, 1], dl[:, 2], dl[:, 3], sc)
    out = _nms()(dec, l1)
    return out.reshape(6, _KPAD)[:, :_TOP_K].T[None, :, :]
